# baseline (speedup 1.0000x reference)
import functools

import jax
import jax.numpy as jnp
from jax import lax
from jax.experimental import pallas as pl
from jax.experimental.pallas import tpu as pltpu
from jax.experimental.pallas import tpu_sc as plsc

N = 10000
D = 256
H = 128
E = 160000
E_PAD = 163840
CH = 128
CHUNKS = E_PAD // (16 * CH)
PER_TILE = E_PAD // 16
ACC_ROWS = 10112
ZROWS_PER_TILE = ACC_ROWS // 16
OUT_ROWS_PER_TILE = 632
OUT_TAIL = N - 15 * OUT_ROWS_PER_TILE
G = 64
R = 400
GRID = N // R

_f32 = jnp.float32


def _sc_agg_body(h0, h1, srcp, dstp, out0, out1,
                 sidx0, didx0, sidx1, didx1, sidx2, didx2, sidx3, didx3,
                 rows0, rows1, acc,
                 semi0, semi1, semi2, semi3, semg0, semg1, sems0, sems1):
    c = lax.axis_index("c")
    s = lax.axis_index("s")

    @pl.loop(0, CH)
    def _zr(i):
        @pl.loop(0, 8)
        def _zc(j):
            rows0[i, pl.ds(j * 16, 16)] = jnp.zeros((16,), _f32)

    zbase = s * ZROWS_PER_TILE
    for k in range(4):
        pltpu.sync_copy(rows0.at[pl.ds(0, CH)],
                        acc.at[pl.ds(zbase + CH * k, CH)])
    pltpu.sync_copy(rows0.at[pl.ds(0, ZROWS_PER_TILE - 4 * CH)],
                    acc.at[pl.ds(zbase + 4 * CH, ZROWS_PER_TILE - 4 * CH)])
    plsc.subcore_barrier()

    ebase = s * PER_TILE
    ibufs = ((sidx0, didx0, semi0), (sidx1, didx1, semi1),
             (sidx2, didx2, semi2), (sidx3, didx3, semi3))
    rbufs = ((rows0, semg0, sems0), (rows1, semg1, sems1))

    def _idx_start(i, p):
        sb, db, semi = ibufs[p]
        off = ebase + i * CH
        pltpu.async_copy(srcp.at[pl.ds(off, CH)], sb, semi)
        pltpu.async_copy(dstp.at[pl.ds(off, CH)], db, semi)

    def _idx_wait(i, p):
        sb, db, semi = ibufs[p]
        off = ebase + i * CH
        pltpu.make_async_copy(srcp.at[pl.ds(off, CH)], sb, semi).wait()
        pltpu.make_async_copy(dstp.at[pl.ds(off, CH)], db, semi).wait()

    def _run(table):
        for p in range(4):
            _idx_start(p, p)
        _idx_wait(0, 0)
        pltpu.async_copy(table.at[sidx0], rows0, semg0)

        @pl.loop(0, CHUNKS // 4)
        def _quad(g):
            for p in range(4):
                b = p % 2
                rb, semg, sems = rbufs[b]
                nrb, nsemg, nsems = rbufs[1 - b]
                i = 4 * g + p
                sb, db, _ = ibufs[p]

                pltpu.make_async_copy(table.at[sb], rb, semg).wait()
                pltpu.async_copy(rb, acc.at[db], sems, add=True)

                @pl.when(i > 0)
                def _():
                    osb, odb, _ = ibufs[(p - 1) % 4]
                    pltpu.make_async_copy(nrb, acc.at[odb], nsems).wait()

                @pl.when(jnp.logical_and(i > 0, i + 3 < CHUNKS))
                def _():
                    _idx_start(i + 3, (p - 1) % 4)

                @pl.when(i + 1 < CHUNKS)
                def _():
                    nsb, _, _ = ibufs[(p + 1) % 4]
                    _idx_wait(i + 1, (p + 1) % 4)
                    pltpu.async_copy(table.at[nsb], nrb, nsemg)

        lsb, ldb, _ = ibufs[(CHUNKS - 1) % 4]
        lrb, _, lsems = rbufs[(CHUNKS - 1) % 2]
        pltpu.make_async_copy(lrb, acc.at[ldb], lsems).wait()

    @pl.when(c == 0)
    def _():
        _run(h0)

    @pl.when(c == 1)
    def _():
        _run(h1)

    plsc.subcore_barrier()
    rbase = s * OUT_ROWS_PER_TILE

    def _copy_out(out):
        @pl.when(s < 15)
        def _():
            pltpu.sync_copy(acc.at[pl.ds(rbase, OUT_ROWS_PER_TILE)],
                            out.at[pl.ds(rbase, OUT_ROWS_PER_TILE)])

        @pl.when(s == 15)
        def _():
            pltpu.sync_copy(acc.at[pl.ds(15 * OUT_ROWS_PER_TILE, OUT_TAIL)],
                            out.at[pl.ds(15 * OUT_ROWS_PER_TILE, OUT_TAIL)])

    @pl.when(c == 0)
    def _():
        _copy_out(out0)

    @pl.when(c == 1)
    def _():
        _copy_out(out1)


@functools.lru_cache(maxsize=None)
def _get_sc_agg():
    mesh = plsc.VectorSubcoreMesh(
        core_axis_name="c", subcore_axis_name="s",
        num_cores=2, num_subcores=16)
    return pl.kernel(
        _sc_agg_body,
        out_type=[jax.ShapeDtypeStruct((N, H), _f32),
                  jax.ShapeDtypeStruct((N, H), _f32)],
        mesh=mesh,
        scratch_types=(
            [pltpu.VMEM((CH,), jnp.int32)] * 8
            + [pltpu.VMEM((CH, H), _f32)] * 2
            + [pltpu.VMEM_SHARED((ACC_ROWS, H), _f32)]
            + [pltpu.SemaphoreType.DMA] * 8
        ),
    )


def _gelu(y):
    return 0.5 * y * (1.0 + lax.erf(y * 0.7071067811865476))


def _layer_a_body(h0_ref, h1_ref, a0_ref, a1_ref, wr_ref, br_ref, w1_ref,
                  b1_ref, w2_ref, b2_ref, eps_ref, h2_ref, id_ref, st_ref):
    i = pl.program_id(0)
    e1 = 1.0 + eps_ref[0, 0]
    h0 = h0_ref[...]
    h1 = h1_ref[...]
    u0 = e1 * h0 + a0_ref[...]
    u1 = e1 * h1 + a1_ref[...]
    ident = (jnp.dot(h0, wr_ref[0:H, :], preferred_element_type=_f32)
             + jnp.dot(h1, wr_ref[H:D, :], preferred_element_type=_f32)
             + br_ref[...])
    t = (jnp.dot(u0, w1_ref[0:H, :], preferred_element_type=_f32)
         + jnp.dot(u1, w1_ref[H:D, :], preferred_element_type=_f32)
         + b1_ref[...])
    t = jnp.maximum(t, 0.0)
    h2 = jnp.dot(t, w2_ref[...], preferred_element_type=_f32) + b2_ref[...]
    h2_ref[...] = h2
    id_ref[...] = ident

    @pl.when(i == 0)
    def _():
        st_ref[...] = jnp.zeros((2, D), _f32)

    st_ref[0:1, :] += jnp.sum(h2, axis=0, keepdims=True)
    st_ref[1:2, :] += jnp.sum(h2 * h2, axis=0, keepdims=True)


def _layer_a(h0, h1, a0, a1, wr, br, w1, b1, w2, b2, eps):
    blk = lambda w: pl.BlockSpec((R, w), lambda i: (i, 0))
    full = lambda a, b: pl.BlockSpec((a, b), lambda i: (0, 0))
    return pl.pallas_call(
        _layer_a_body,
        grid=(GRID,),
        in_specs=[blk(H), blk(H), blk(H), blk(H),
                  full(D, D), full(1, D), full(D, D), full(1, D),
                  full(D, D), full(1, D), full(1, 1)],
        out_specs=[blk(D), blk(D), full(2, D)],
        out_shape=[jax.ShapeDtypeStruct((N, D), _f32),
                   jax.ShapeDtypeStruct((N, D), _f32),
                   jax.ShapeDtypeStruct((2, D), _f32)],
    )(h0, h1, a0, a1, wr, br, w1, b1, w2, b2, eps)


def _layer_b_body(h2_ref, id_ref, st_ref, g_ref, be_ref, o0_ref, o1_ref):
    st = st_ref[...]
    mu = st[0:1, :] * (1.0 / N)
    var = st[1:2, :] * (1.0 / N) - mu * mu
    inv = lax.rsqrt(var + 1e-5) * g_ref[...]
    y = (h2_ref[...] - mu) * inv + be_ref[...] + id_ref[...]
    y = _gelu(y)
    o0_ref[...] = y[:, 0:H]
    o1_ref[...] = y[:, H:D]


def _layer_b(h2, ident, st, gamma, beta):
    blk = lambda w: pl.BlockSpec((R, w), lambda i: (i, 0))
    full = lambda a, b: pl.BlockSpec((a, b), lambda i: (0, 0))
    return pl.pallas_call(
        _layer_b_body,
        grid=(GRID,),
        in_specs=[blk(D), blk(D), full(2, D), full(1, D), full(1, D)],
        out_specs=[blk(H), blk(H)],
        out_shape=[jax.ShapeDtypeStruct((N, H), _f32),
                   jax.ShapeDtypeStruct((N, H), _f32)],
    )(h2, ident, st, gamma, beta)


def _head_body(h0_ref, h1_ref, b_ref, wf1_ref, bf1_ref, wf2_ref, bf2_ref,
               out_ref, sums, cntm):
    i = pl.program_id(0)

    @pl.when(i == 0)
    def _():
        sums[...] = jnp.zeros((G, D), _f32)
        cntm[...] = jnp.zeros((G, H), _f32)

    oh = (b_ref[...] == lax.broadcasted_iota(jnp.int32, (R, G), 1)).astype(_f32)
    dn = (((0,), (0,)), ((), ()))
    sums[:, 0:H] += lax.dot_general(oh, h0_ref[...], dn,
                                    preferred_element_type=_f32)
    sums[:, H:D] += lax.dot_general(oh, h1_ref[...], dn,
                                    preferred_element_type=_f32)
    cntm[...] += lax.dot_general(oh, jnp.ones((R, H), _f32), dn,
                                 preferred_element_type=_f32)

    @pl.when(i == GRID - 1)
    def _():
        cnt = jnp.maximum(cntm[...], 1.0)
        p0 = sums[:, 0:H] / cnt
        p1 = sums[:, H:D] / cnt
        z = (jnp.dot(p0, wf1_ref[0:H, :], preferred_element_type=_f32)
             + jnp.dot(p1, wf1_ref[H:D, :], preferred_element_type=_f32)
             + bf1_ref[...])
        z = _gelu(z)
        out_ref[...] = (jnp.dot(z, wf2_ref[...], preferred_element_type=_f32)
                        + bf2_ref[...])


def _head(h0, h1, batch2, wf1, bf1, wf2, bf2):
    blk = lambda w: pl.BlockSpec((R, w), lambda i: (i, 0))
    full = lambda a, b: pl.BlockSpec((a, b), lambda i: (0, 0))
    return pl.pallas_call(
        _head_body,
        grid=(GRID,),
        in_specs=[blk(H), blk(H), blk(1),
                  full(D, D), full(1, D), full(D, 10), full(1, 10)],
        out_specs=pl.BlockSpec((G, 10), lambda i: (0, 0)),
        out_shape=jax.ShapeDtypeStruct((G, 10), _f32),
        scratch_shapes=[pltpu.VMEM((G, D), _f32), pltpu.VMEM((G, H), _f32)],
    )(h0, h1, batch2, wf1, bf1, wf2, bf2)


def kernel(x, edge_index, batch, params):
    src = edge_index[0]
    dst = edge_index[1]
    npad = E_PAD - E
    ar = jnp.arange(npad, dtype=jnp.int32)
    srcp = jnp.concatenate([src, (ar * 997) % N])
    dstp = jnp.concatenate([dst, N + (ar % 16)])
    batch2 = batch.reshape(N, 1)

    h0 = x[:, 0:H]
    h1 = x[:, H:D]
    for l in range(4):
        g = params[f"gin{l}"]
        bn = params[f"bn{l}"]
        rs = params[f"res{l}"]
        a0, a1 = _get_sc_agg()(h0, h1, srcp, dstp)
        h2, ident, st = _layer_a(
            h0, h1, a0, a1, rs["W"], rs["b"].reshape(1, D),
            g["lin1"]["W"], g["lin1"]["b"].reshape(1, D),
            g["lin2"]["W"], g["lin2"]["b"].reshape(1, D),
            g["eps"].reshape(1, 1))
        h0, h1 = _layer_b(h2, ident, st, bn["gamma"].reshape(1, D),
                          bn["beta"].reshape(1, D))
    return _head(h0, h1, batch2, params["fc1"]["W"],
                 params["fc1"]["b"].reshape(1, D), params["fc2"]["W"],
                 params["fc2"]["b"].reshape(1, 10))


# speedup vs baseline: 1.1366x; 1.1366x over previous
import functools

import jax
import jax.numpy as jnp
from jax import lax
from jax.experimental import pallas as pl
from jax.experimental.pallas import tpu as pltpu
from jax.experimental.pallas import tpu_sc as plsc

N = 10000
D = 256
H = 128
E = 160000
E_PAD = 163840
CH = 128
CHUNKS = E_PAD // (16 * CH)
PER_TILE = E_PAD // 16
ACC_ROWS = 10112
ZROWS_PER_TILE = ACC_ROWS // 16
OUT_ROWS_PER_TILE = 632
OUT_TAIL = N - 15 * OUT_ROWS_PER_TILE
G = 64
R = 400
GRID = N // R

_f32 = jnp.float32


def _sc_agg_body(h0, h1, srcp, dstp, out0, out1,
                 sidx0, didx0, sidx1, didx1, sidx2, didx2, sidx3, didx3,
                 rows0, rows1, acc,
                 semi0, semi1, semi2, semi3, semg0, semg1, sems0, sems1):
    c = lax.axis_index("c")
    s = lax.axis_index("s")

    @pl.loop(0, CH)
    def _zr(i):
        @pl.loop(0, 8)
        def _zc(j):
            rows0[i, pl.ds(j * 16, 16)] = jnp.zeros((16,), _f32)

    zbase = s * ZROWS_PER_TILE
    for k in range(4):
        pltpu.sync_copy(rows0.at[pl.ds(0, CH)],
                        acc.at[pl.ds(zbase + CH * k, CH)])
    pltpu.sync_copy(rows0.at[pl.ds(0, ZROWS_PER_TILE - 4 * CH)],
                    acc.at[pl.ds(zbase + 4 * CH, ZROWS_PER_TILE - 4 * CH)])
    plsc.subcore_barrier()

    ebase = s * PER_TILE
    ibufs = ((sidx0, didx0, semi0), (sidx1, didx1, semi1),
             (sidx2, didx2, semi2), (sidx3, didx3, semi3))
    rbufs = ((rows0, semg0, sems0), (rows1, semg1, sems1))

    def _idx_start(i, p):
        sb, db, semi = ibufs[p]
        off = ebase + i * CH
        pltpu.async_copy(srcp.at[pl.ds(off, CH)], sb, semi)
        pltpu.async_copy(dstp.at[pl.ds(off, CH)], db, semi)

    def _idx_wait(i, p):
        sb, db, semi = ibufs[p]
        off = ebase + i * CH
        pltpu.make_async_copy(srcp.at[pl.ds(off, CH)], sb, semi).wait()
        pltpu.make_async_copy(dstp.at[pl.ds(off, CH)], db, semi).wait()

    def _run(table):
        for p in range(4):
            _idx_start(p, p)
        _idx_wait(0, 0)
        pltpu.async_copy(table.at[sidx0], rows0, semg0)

        @pl.loop(0, CHUNKS // 4)
        def _quad(g):
            for p in range(4):
                b = p % 2
                rb, semg, _ = rbufs[b]
                nrb, nsemg, _ = rbufs[1 - b]
                i = 4 * g + p
                sb, db, _ = ibufs[p]

                @pl.when(i + 1 < CHUNKS)
                def _():
                    nsb, _, _ = ibufs[(p + 1) % 4]
                    _idx_wait(i + 1, (p + 1) % 4)
                    pltpu.async_copy(table.at[nsb], nrb, nsemg)

                pltpu.make_async_copy(table.at[sb], rb, semg).wait()
                pltpu.sync_copy(rb, acc.at[db], add=True)

                @pl.when(i + 4 < CHUNKS)
                def _():
                    _idx_start(i + 4, p)

    @pl.when(c == 0)
    def _():
        _run(h0)

    @pl.when(c == 1)
    def _():
        _run(h1)

    plsc.subcore_barrier()
    rbase = s * OUT_ROWS_PER_TILE

    def _copy_out(out):
        @pl.when(s < 15)
        def _():
            pltpu.sync_copy(acc.at[pl.ds(rbase, OUT_ROWS_PER_TILE)],
                            out.at[pl.ds(rbase, OUT_ROWS_PER_TILE)])

        @pl.when(s == 15)
        def _():
            pltpu.sync_copy(acc.at[pl.ds(15 * OUT_ROWS_PER_TILE, OUT_TAIL)],
                            out.at[pl.ds(15 * OUT_ROWS_PER_TILE, OUT_TAIL)])

    @pl.when(c == 0)
    def _():
        _copy_out(out0)

    @pl.when(c == 1)
    def _():
        _copy_out(out1)


@functools.lru_cache(maxsize=None)
def _get_sc_agg():
    mesh = plsc.VectorSubcoreMesh(
        core_axis_name="c", subcore_axis_name="s",
        num_cores=2, num_subcores=16)
    return pl.kernel(
        _sc_agg_body,
        out_type=[jax.ShapeDtypeStruct((N, H), _f32),
                  jax.ShapeDtypeStruct((N, H), _f32)],
        mesh=mesh,
        scratch_types=(
            [pltpu.VMEM((CH,), jnp.int32)] * 8
            + [pltpu.VMEM((CH, H), _f32)] * 2
            + [pltpu.VMEM_SHARED((ACC_ROWS, H), _f32)]
            + [pltpu.SemaphoreType.DMA] * 8
        ),
    )


def _gelu(y):
    return 0.5 * y * (1.0 + lax.erf(y * 0.7071067811865476))


def _layer_a_body(h0_ref, h1_ref, a0_ref, a1_ref, wr_ref, br_ref, w1_ref,
                  b1_ref, w2_ref, b2_ref, eps_ref, h2_ref, id_ref, st_ref):
    i = pl.program_id(0)
    e1 = 1.0 + eps_ref[0, 0]
    h0 = h0_ref[...]
    h1 = h1_ref[...]
    u0 = e1 * h0 + a0_ref[...]
    u1 = e1 * h1 + a1_ref[...]
    ident = (jnp.dot(h0, wr_ref[0:H, :], preferred_element_type=_f32)
             + jnp.dot(h1, wr_ref[H:D, :], preferred_element_type=_f32)
             + br_ref[...])
    t = (jnp.dot(u0, w1_ref[0:H, :], preferred_element_type=_f32)
         + jnp.dot(u1, w1_ref[H:D, :], preferred_element_type=_f32)
         + b1_ref[...])
    t = jnp.maximum(t, 0.0)
    h2 = jnp.dot(t, w2_ref[...], preferred_element_type=_f32) + b2_ref[...]
    h2_ref[...] = h2
    id_ref[...] = ident

    @pl.when(i == 0)
    def _():
        st_ref[...] = jnp.zeros((2, D), _f32)

    st_ref[0:1, :] += jnp.sum(h2, axis=0, keepdims=True)
    st_ref[1:2, :] += jnp.sum(h2 * h2, axis=0, keepdims=True)


def _layer_a(h0, h1, a0, a1, wr, br, w1, b1, w2, b2, eps):
    blk = lambda w: pl.BlockSpec((R, w), lambda i: (i, 0))
    full = lambda a, b: pl.BlockSpec((a, b), lambda i: (0, 0))
    return pl.pallas_call(
        _layer_a_body,
        grid=(GRID,),
        in_specs=[blk(H), blk(H), blk(H), blk(H),
                  full(D, D), full(1, D), full(D, D), full(1, D),
                  full(D, D), full(1, D), full(1, 1)],
        out_specs=[blk(D), blk(D), full(2, D)],
        out_shape=[jax.ShapeDtypeStruct((N, D), _f32),
                   jax.ShapeDtypeStruct((N, D), _f32),
                   jax.ShapeDtypeStruct((2, D), _f32)],
    )(h0, h1, a0, a1, wr, br, w1, b1, w2, b2, eps)


def _layer_b_body(h2_ref, id_ref, st_ref, g_ref, be_ref, o0_ref, o1_ref):
    st = st_ref[...]
    mu = st[0:1, :] * (1.0 / N)
    var = st[1:2, :] * (1.0 / N) - mu * mu
    inv = lax.rsqrt(var + 1e-5) * g_ref[...]
    y = (h2_ref[...] - mu) * inv + be_ref[...] + id_ref[...]
    y = _gelu(y)
    o0_ref[...] = y[:, 0:H]
    o1_ref[...] = y[:, H:D]


def _layer_b(h2, ident, st, gamma, beta):
    blk = lambda w: pl.BlockSpec((R, w), lambda i: (i, 0))
    full = lambda a, b: pl.BlockSpec((a, b), lambda i: (0, 0))
    return pl.pallas_call(
        _layer_b_body,
        grid=(GRID,),
        in_specs=[blk(D), blk(D), full(2, D), full(1, D), full(1, D)],
        out_specs=[blk(H), blk(H)],
        out_shape=[jax.ShapeDtypeStruct((N, H), _f32),
                   jax.ShapeDtypeStruct((N, H), _f32)],
    )(h2, ident, st, gamma, beta)


def _head_body(h0_ref, h1_ref, b_ref, wf1_ref, bf1_ref, wf2_ref, bf2_ref,
               out_ref, sums, cntm):
    i = pl.program_id(0)

    @pl.when(i == 0)
    def _():
        sums[...] = jnp.zeros((G, D), _f32)
        cntm[...] = jnp.zeros((G, H), _f32)

    oh = (b_ref[...] == lax.broadcasted_iota(jnp.int32, (R, G), 1)).astype(_f32)
    dn = (((0,), (0,)), ((), ()))
    sums[:, 0:H] += lax.dot_general(oh, h0_ref[...], dn,
                                    preferred_element_type=_f32)
    sums[:, H:D] += lax.dot_general(oh, h1_ref[...], dn,
                                    preferred_element_type=_f32)
    cntm[...] += lax.dot_general(oh, jnp.ones((R, H), _f32), dn,
                                 preferred_element_type=_f32)

    @pl.when(i == GRID - 1)
    def _():
        cnt = jnp.maximum(cntm[...], 1.0)
        p0 = sums[:, 0:H] / cnt
        p1 = sums[:, H:D] / cnt
        z = (jnp.dot(p0, wf1_ref[0:H, :], preferred_element_type=_f32)
             + jnp.dot(p1, wf1_ref[H:D, :], preferred_element_type=_f32)
             + bf1_ref[...])
        z = _gelu(z)
        out_ref[...] = (jnp.dot(z, wf2_ref[...], preferred_element_type=_f32)
                        + bf2_ref[...])


def _head(h0, h1, batch2, wf1, bf1, wf2, bf2):
    blk = lambda w: pl.BlockSpec((R, w), lambda i: (i, 0))
    full = lambda a, b: pl.BlockSpec((a, b), lambda i: (0, 0))
    return pl.pallas_call(
        _head_body,
        grid=(GRID,),
        in_specs=[blk(H), blk(H), blk(1),
                  full(D, D), full(1, D), full(D, 10), full(1, 10)],
        out_specs=pl.BlockSpec((G, 10), lambda i: (0, 0)),
        out_shape=jax.ShapeDtypeStruct((G, 10), _f32),
        scratch_shapes=[pltpu.VMEM((G, D), _f32), pltpu.VMEM((G, H), _f32)],
    )(h0, h1, batch2, wf1, bf1, wf2, bf2)


def kernel(x, edge_index, batch, params):
    src = edge_index[0]
    dst = edge_index[1]
    npad = E_PAD - E
    ar = jnp.arange(npad, dtype=jnp.int32)
    srcp = jnp.concatenate([src, (ar * 997) % N])
    dstp = jnp.concatenate([dst, N + (ar % 16)])
    batch2 = batch.reshape(N, 1)

    h0 = x[:, 0:H]
    h1 = x[:, H:D]
    for l in range(4):
        g = params[f"gin{l}"]
        bn = params[f"bn{l}"]
        rs = params[f"res{l}"]
        a0, a1 = _get_sc_agg()(h0, h1, srcp, dstp)
        h2, ident, st = _layer_a(
            h0, h1, a0, a1, rs["W"], rs["b"].reshape(1, D),
            g["lin1"]["W"], g["lin1"]["b"].reshape(1, D),
            g["lin2"]["W"], g["lin2"]["b"].reshape(1, D),
            g["eps"].reshape(1, 1))
        h0, h1 = _layer_b(h2, ident, st, bn["gamma"].reshape(1, D),
                          bn["beta"].reshape(1, D))
    return _head(h0, h1, batch2, params["fc1"]["W"],
                 params["fc1"]["b"].reshape(1, D), params["fc2"]["W"],
                 params["fc2"]["b"].reshape(1, 10))


# speedup vs baseline: 1.1564x; 1.0174x over previous
import functools

import jax
import jax.numpy as jnp
from jax import lax
from jax.experimental import pallas as pl
from jax.experimental.pallas import tpu as pltpu
from jax.experimental.pallas import tpu_sc as plsc

N = 10000
D = 256
H = 128
E = 160000
E_PAD = 163840
CH = 128
CHUNKS = E_PAD // (16 * CH)
PER_TILE = E_PAD // 16
ACC_ROWS = 10112
ZROWS_PER_TILE = ACC_ROWS // 16
OUT_ROWS_PER_TILE = 632
OUT_TAIL = N - 15 * OUT_ROWS_PER_TILE
G = 64
R = 400
GRID = N // R

_f32 = jnp.float32


def _sc_agg_body(h0, h1, srcp, dstp, out0, out1,
                 sidx0, didx0, sidx1, didx1, sidx2, didx2, sidx3, didx3,
                 rows0, rows1, acc,
                 semi0, semi1, semi2, semi3, semg0, semg1, sems0, sems1):
    c = lax.axis_index("c")
    s = lax.axis_index("s")

    @pl.loop(0, CH)
    def _zr(i):
        @pl.loop(0, 8)
        def _zc(j):
            rows0[i, pl.ds(j * 16, 16)] = jnp.zeros((16,), _f32)

    zbase = s * ZROWS_PER_TILE
    for k in range(4):
        pltpu.sync_copy(rows0.at[pl.ds(0, CH)],
                        acc.at[pl.ds(zbase + CH * k, CH)])
    pltpu.sync_copy(rows0.at[pl.ds(0, ZROWS_PER_TILE - 4 * CH)],
                    acc.at[pl.ds(zbase + 4 * CH, ZROWS_PER_TILE - 4 * CH)])
    plsc.subcore_barrier()

    ebase = s * PER_TILE
    ibufs = ((sidx0, didx0, semi0), (sidx1, didx1, semi1),
             (sidx2, didx2, semi2), (sidx3, didx3, semi3))
    rbufs = ((rows0, semg0, sems0), (rows1, semg1, sems1))

    def _idx_start(i, p):
        sb, db, semi = ibufs[p]
        off = ebase + i * CH
        pltpu.async_copy(srcp.at[pl.ds(off, CH)], sb, semi)
        pltpu.async_copy(dstp.at[pl.ds(off, CH)], db, semi)

    def _idx_wait(i, p):
        sb, db, semi = ibufs[p]
        off = ebase + i * CH
        pltpu.make_async_copy(srcp.at[pl.ds(off, CH)], sb, semi).wait()
        pltpu.make_async_copy(dstp.at[pl.ds(off, CH)], db, semi).wait()

    def _run(table):
        for p in range(4):
            _idx_start(p, p)
        _idx_wait(0, 0)
        pltpu.async_copy(table.at[sidx0], rows0, semg0)

        @pl.loop(0, CHUNKS // 4)
        def _quad(g):
            for p in range(4):
                b = p % 2
                rb, semg, _ = rbufs[b]
                nrb, nsemg, _ = rbufs[1 - b]
                i = 4 * g + p
                sb, db, _ = ibufs[p]

                @pl.when(i + 1 < CHUNKS)
                def _():
                    nsb, _, _ = ibufs[(p + 1) % 4]
                    _idx_wait(i + 1, (p + 1) % 4)
                    pltpu.async_copy(table.at[nsb], nrb, nsemg)

                pltpu.make_async_copy(table.at[sb], rb, semg).wait()
                pltpu.sync_copy(rb, acc.at[db], add=True)

                @pl.when(i + 4 < CHUNKS)
                def _():
                    _idx_start(i + 4, p)

    @pl.when(c == 0)
    def _():
        _run(h0)

    @pl.when(c == 1)
    def _():
        _run(h1)

    plsc.subcore_barrier()
    rbase = s * OUT_ROWS_PER_TILE

    def _copy_out(out):
        @pl.when(s < 15)
        def _():
            pltpu.sync_copy(acc.at[pl.ds(rbase, OUT_ROWS_PER_TILE)],
                            out.at[pl.ds(rbase, OUT_ROWS_PER_TILE)])

        @pl.when(s == 15)
        def _():
            pltpu.sync_copy(acc.at[pl.ds(15 * OUT_ROWS_PER_TILE, OUT_TAIL)],
                            out.at[pl.ds(15 * OUT_ROWS_PER_TILE, OUT_TAIL)])

    @pl.when(c == 0)
    def _():
        _copy_out(out0)

    @pl.when(c == 1)
    def _():
        _copy_out(out1)


@functools.lru_cache(maxsize=None)
def _get_sc_agg():
    mesh = plsc.VectorSubcoreMesh(
        core_axis_name="c", subcore_axis_name="s",
        num_cores=2, num_subcores=16)
    return pl.kernel(
        _sc_agg_body,
        out_type=[jax.ShapeDtypeStruct((N, H), _f32),
                  jax.ShapeDtypeStruct((N, H), _f32)],
        mesh=mesh,
        scratch_types=(
            [pltpu.VMEM((CH,), jnp.int32)] * 8
            + [pltpu.VMEM((CH, H), _f32)] * 2
            + [pltpu.VMEM_SHARED((ACC_ROWS, H), _f32)]
            + [pltpu.SemaphoreType.DMA] * 8
        ),
    )


def _gelu(y):
    return 0.5 * y * (1.0 + lax.erf(y * 0.7071067811865476))


def _ident_body(h0_ref, h1_ref, wr_ref, br_ref, id_ref):
    id_ref[...] = (jnp.dot(h0_ref[...], wr_ref[0:H, :],
                           preferred_element_type=_f32)
                   + jnp.dot(h1_ref[...], wr_ref[H:D, :],
                             preferred_element_type=_f32)
                   + br_ref[...])


def _ident(h0, h1, wr, br):
    blk = lambda w: pl.BlockSpec((R, w), lambda i: (i, 0))
    full = lambda a, b: pl.BlockSpec((a, b), lambda i: (0, 0))
    return pl.pallas_call(
        _ident_body,
        grid=(GRID,),
        in_specs=[blk(H), blk(H), full(D, D), full(1, D)],
        out_specs=blk(D),
        out_shape=jax.ShapeDtypeStruct((N, D), _f32),
    )(h0, h1, wr, br)


def _layer_a_body(h0_ref, h1_ref, a0_ref, a1_ref, w1_ref,
                  b1_ref, w2_ref, b2_ref, eps_ref, h2_ref, st_ref):
    i = pl.program_id(0)
    e1 = 1.0 + eps_ref[0, 0]
    u0 = e1 * h0_ref[...] + a0_ref[...]
    u1 = e1 * h1_ref[...] + a1_ref[...]
    t = (jnp.dot(u0, w1_ref[0:H, :], preferred_element_type=_f32)
         + jnp.dot(u1, w1_ref[H:D, :], preferred_element_type=_f32)
         + b1_ref[...])
    t = jnp.maximum(t, 0.0)
    h2 = jnp.dot(t, w2_ref[...], preferred_element_type=_f32) + b2_ref[...]
    h2_ref[...] = h2

    @pl.when(i == 0)
    def _():
        st_ref[...] = jnp.zeros((2, D), _f32)

    st_ref[0:1, :] += jnp.sum(h2, axis=0, keepdims=True)
    st_ref[1:2, :] += jnp.sum(h2 * h2, axis=0, keepdims=True)


def _layer_a(h0, h1, a0, a1, w1, b1, w2, b2, eps):
    blk = lambda w: pl.BlockSpec((R, w), lambda i: (i, 0))
    full = lambda a, b: pl.BlockSpec((a, b), lambda i: (0, 0))
    return pl.pallas_call(
        _layer_a_body,
        grid=(GRID,),
        in_specs=[blk(H), blk(H), blk(H), blk(H),
                  full(D, D), full(1, D),
                  full(D, D), full(1, D), full(1, 1)],
        out_specs=[blk(D), full(2, D)],
        out_shape=[jax.ShapeDtypeStruct((N, D), _f32),
                   jax.ShapeDtypeStruct((2, D), _f32)],
    )(h0, h1, a0, a1, w1, b1, w2, b2, eps)


def _layer_b_body(h2_ref, id_ref, st_ref, g_ref, be_ref, o0_ref, o1_ref):
    st = st_ref[...]
    mu = st[0:1, :] * (1.0 / N)
    var = st[1:2, :] * (1.0 / N) - mu * mu
    inv = lax.rsqrt(var + 1e-5) * g_ref[...]
    y = (h2_ref[...] - mu) * inv + be_ref[...] + id_ref[...]
    y = _gelu(y)
    o0_ref[...] = y[:, 0:H]
    o1_ref[...] = y[:, H:D]


def _layer_b(h2, ident, st, gamma, beta):
    blk = lambda w: pl.BlockSpec((R, w), lambda i: (i, 0))
    full = lambda a, b: pl.BlockSpec((a, b), lambda i: (0, 0))
    return pl.pallas_call(
        _layer_b_body,
        grid=(GRID,),
        in_specs=[blk(D), blk(D), full(2, D), full(1, D), full(1, D)],
        out_specs=[blk(H), blk(H)],
        out_shape=[jax.ShapeDtypeStruct((N, H), _f32),
                   jax.ShapeDtypeStruct((N, H), _f32)],
    )(h2, ident, st, gamma, beta)


def _head_body(h0_ref, h1_ref, b_ref, wf1_ref, bf1_ref, wf2_ref, bf2_ref,
               out_ref, sums, cntm):
    i = pl.program_id(0)

    @pl.when(i == 0)
    def _():
        sums[...] = jnp.zeros((G, D), _f32)
        cntm[...] = jnp.zeros((G, H), _f32)

    oh = (b_ref[...] == lax.broadcasted_iota(jnp.int32, (R, G), 1)).astype(_f32)
    dn = (((0,), (0,)), ((), ()))
    sums[:, 0:H] += lax.dot_general(oh, h0_ref[...], dn,
                                    preferred_element_type=_f32)
    sums[:, H:D] += lax.dot_general(oh, h1_ref[...], dn,
                                    preferred_element_type=_f32)
    cntm[...] += lax.dot_general(oh, jnp.ones((R, H), _f32), dn,
                                 preferred_element_type=_f32)

    @pl.when(i == GRID - 1)
    def _():
        cnt = jnp.maximum(cntm[...], 1.0)
        p0 = sums[:, 0:H] / cnt
        p1 = sums[:, H:D] / cnt
        z = (jnp.dot(p0, wf1_ref[0:H, :], preferred_element_type=_f32)
             + jnp.dot(p1, wf1_ref[H:D, :], preferred_element_type=_f32)
             + bf1_ref[...])
        z = _gelu(z)
        out_ref[...] = (jnp.dot(z, wf2_ref[...], preferred_element_type=_f32)
                        + bf2_ref[...])


def _head(h0, h1, batch2, wf1, bf1, wf2, bf2):
    blk = lambda w: pl.BlockSpec((R, w), lambda i: (i, 0))
    full = lambda a, b: pl.BlockSpec((a, b), lambda i: (0, 0))
    return pl.pallas_call(
        _head_body,
        grid=(GRID,),
        in_specs=[blk(H), blk(H), blk(1),
                  full(D, D), full(1, D), full(D, 10), full(1, 10)],
        out_specs=pl.BlockSpec((G, 10), lambda i: (0, 0)),
        out_shape=jax.ShapeDtypeStruct((G, 10), _f32),
        scratch_shapes=[pltpu.VMEM((G, D), _f32), pltpu.VMEM((G, H), _f32)],
    )(h0, h1, batch2, wf1, bf1, wf2, bf2)


def kernel(x, edge_index, batch, params):
    src = edge_index[0]
    dst = edge_index[1]
    npad = E_PAD - E
    ar = jnp.arange(npad, dtype=jnp.int32)
    srcp = jnp.concatenate([src, (ar * 997) % N])
    dstp = jnp.concatenate([dst, N + (ar % 16)])
    batch2 = batch.reshape(N, 1)

    h0 = x[:, 0:H]
    h1 = x[:, H:D]
    for l in range(4):
        g = params[f"gin{l}"]
        bn = params[f"bn{l}"]
        rs = params[f"res{l}"]
        a0, a1 = _get_sc_agg()(h0, h1, srcp, dstp)
        ident = _ident(h0, h1, rs["W"], rs["b"].reshape(1, D))
        h2, st = _layer_a(
            h0, h1, a0, a1,
            g["lin1"]["W"], g["lin1"]["b"].reshape(1, D),
            g["lin2"]["W"], g["lin2"]["b"].reshape(1, D),
            g["eps"].reshape(1, 1))
        h0, h1 = _layer_b(h2, ident, st, bn["gamma"].reshape(1, D),
                          bn["beta"].reshape(1, D))
    return _head(h0, h1, batch2, params["fc1"]["W"],
                 params["fc1"]["b"].reshape(1, D), params["fc2"]["W"],
                 params["fc2"]["b"].reshape(1, 10))


# speedup vs baseline: 1.1833x; 1.0233x over previous
import functools

import jax
import jax.numpy as jnp
from jax import lax
from jax.experimental import pallas as pl
from jax.experimental.pallas import tpu as pltpu
from jax.experimental.pallas import tpu_sc as plsc

N = 10000
D = 256
H = 128
E = 160000
CH = 128
NCHUNK = E // CH
BASE_CHUNKS = NCHUNK // 16
MAX_CHUNKS = BASE_CHUNKS + 1
ACC_ROWS = 10112
ZROWS_PER_TILE = ACC_ROWS // 16
OUT_ROWS_PER_TILE = 632
OUT_TAIL = N - 15 * OUT_ROWS_PER_TILE
G = 64
R = 400
GRID = N // R

_f32 = jnp.float32


def _sc_agg_body(h0, h1, srcp, dstp, out0, out1,
                 sidx0, didx0, sidx1, didx1, sidx2, didx2, sidx3, didx3,
                 rows0, rows1, zbuf, acc,
                 semi0, semi1, semi2, semi3, semg0, semg1, sems0, sems1):
    c = lax.axis_index("c")
    s = lax.axis_index("s")

    climit = BASE_CHUNKS + jnp.where(s < 2, 1, 0)
    ebase = (BASE_CHUNKS * s + jnp.minimum(s, 2)) * CH

    ibufs = ((sidx0, didx0, semi0), (sidx1, didx1, semi1),
             (sidx2, didx2, semi2), (sidx3, didx3, semi3))
    rbufs = ((rows0, semg0, sems0), (rows1, semg1, sems1))

    def _idx_start(i, p):
        sb, db, semi = ibufs[p]
        off = ebase + i * CH
        pltpu.async_copy(srcp.at[pl.ds(off, CH)], sb, semi)
        pltpu.async_copy(dstp.at[pl.ds(off, CH)], db, semi)

    def _idx_wait(i, p):
        sb, db, semi = ibufs[p]
        off = ebase + i * CH
        pltpu.make_async_copy(srcp.at[pl.ds(off, CH)], sb, semi).wait()
        pltpu.make_async_copy(dstp.at[pl.ds(off, CH)], db, semi).wait()

    def _run(table):
        for p in range(4):
            _idx_start(p, p)

        @pl.loop(0, 64)
        def _zr(i):
            @pl.loop(0, 8)
            def _zc(j):
                zbuf[i, pl.ds(j * 16, 16)] = jnp.zeros((16,), _f32)

        _idx_wait(0, 0)
        pltpu.async_copy(table.at[sidx0], rows0, semg0)
        _idx_wait(1, 1)
        pltpu.async_copy(table.at[sidx1], rows1, semg1)

        zbase = s * ZROWS_PER_TILE
        for k in range(9):
            pltpu.sync_copy(zbuf.at[pl.ds(0, 64)],
                            acc.at[pl.ds(zbase + 64 * k, 64)])
        pltpu.sync_copy(zbuf.at[pl.ds(0, ZROWS_PER_TILE - 576)],
                        acc.at[pl.ds(zbase + 576, ZROWS_PER_TILE - 576)])
        plsc.subcore_barrier()

        @pl.loop(0, MAX_CHUNKS // 4 + 1)
        def _quad(g):
            for p in range(4):
                b = p % 2
                rb, semg, _ = rbufs[b]
                nrb, nsemg, _ = rbufs[1 - b]
                i = 4 * g + p
                sb, db, _ = ibufs[p]

                @pl.when(jnp.logical_and(i + 1 < climit, i > 0))
                def _():
                    nsb, _, _ = ibufs[(p + 1) % 4]
                    _idx_wait(i + 1, (p + 1) % 4)
                    pltpu.async_copy(table.at[nsb], nrb, nsemg)

                @pl.when(i < climit)
                def _():
                    pltpu.make_async_copy(table.at[sb], rb, semg).wait()
                    pltpu.sync_copy(rb, acc.at[db], add=True)

                @pl.when(i + 4 < climit)
                def _():
                    _idx_start(i + 4, p)

    @pl.when(c == 0)
    def _():
        _run(h0)

    @pl.when(c == 1)
    def _():
        _run(h1)

    plsc.subcore_barrier()
    rbase = s * OUT_ROWS_PER_TILE

    def _copy_out(out):
        @pl.when(s < 15)
        def _():
            pltpu.sync_copy(acc.at[pl.ds(rbase, OUT_ROWS_PER_TILE)],
                            out.at[pl.ds(rbase, OUT_ROWS_PER_TILE)])

        @pl.when(s == 15)
        def _():
            pltpu.sync_copy(acc.at[pl.ds(15 * OUT_ROWS_PER_TILE, OUT_TAIL)],
                            out.at[pl.ds(15 * OUT_ROWS_PER_TILE, OUT_TAIL)])

    @pl.when(c == 0)
    def _():
        _copy_out(out0)

    @pl.when(c == 1)
    def _():
        _copy_out(out1)


@functools.lru_cache(maxsize=None)
def _get_sc_agg():
    mesh = plsc.VectorSubcoreMesh(
        core_axis_name="c", subcore_axis_name="s",
        num_cores=2, num_subcores=16)
    return pl.kernel(
        _sc_agg_body,
        out_type=[jax.ShapeDtypeStruct((N, H), _f32),
                  jax.ShapeDtypeStruct((N, H), _f32)],
        mesh=mesh,
        scratch_types=(
            [pltpu.VMEM((CH,), jnp.int32)] * 8
            + [pltpu.VMEM((CH, H), _f32)] * 2
            + [pltpu.VMEM((64, H), _f32)]
            + [pltpu.VMEM_SHARED((ACC_ROWS, H), _f32)]
            + [pltpu.SemaphoreType.DMA] * 8
        ),
    )


def _gelu(y):
    return 0.5 * y * (1.0 + lax.erf(y * 0.7071067811865476))


def _ident_body(h0_ref, h1_ref, wr_ref, br_ref, id_ref):
    id_ref[...] = (jnp.dot(h0_ref[...], wr_ref[0:H, :],
                           preferred_element_type=_f32)
                   + jnp.dot(h1_ref[...], wr_ref[H:D, :],
                             preferred_element_type=_f32)
                   + br_ref[...])


def _ident(h0, h1, wr, br):
    blk = lambda w: pl.BlockSpec((R, w), lambda i: (i, 0))
    full = lambda a, b: pl.BlockSpec((a, b), lambda i: (0, 0))
    return pl.pallas_call(
        _ident_body,
        grid=(GRID,),
        in_specs=[blk(H), blk(H), full(D, D), full(1, D)],
        out_specs=blk(D),
        out_shape=jax.ShapeDtypeStruct((N, D), _f32),
    )(h0, h1, wr, br)


def _layer_a_body(h0_ref, h1_ref, a0_ref, a1_ref, w1_ref,
                  b1_ref, w2_ref, b2_ref, eps_ref, h2_ref, st_ref):
    i = pl.program_id(0)
    e1 = 1.0 + eps_ref[0, 0]
    u0 = e1 * h0_ref[...] + a0_ref[...]
    u1 = e1 * h1_ref[...] + a1_ref[...]
    t = (jnp.dot(u0, w1_ref[0:H, :], preferred_element_type=_f32)
         + jnp.dot(u1, w1_ref[H:D, :], preferred_element_type=_f32)
         + b1_ref[...])
    t = jnp.maximum(t, 0.0)
    h2 = jnp.dot(t, w2_ref[...], preferred_element_type=_f32) + b2_ref[...]
    h2_ref[...] = h2

    @pl.when(i == 0)
    def _():
        st_ref[...] = jnp.zeros((2, D), _f32)

    st_ref[0:1, :] += jnp.sum(h2, axis=0, keepdims=True)
    st_ref[1:2, :] += jnp.sum(h2 * h2, axis=0, keepdims=True)


def _layer_a(h0, h1, a0, a1, w1, b1, w2, b2, eps):
    blk = lambda w: pl.BlockSpec((R, w), lambda i: (i, 0))
    full = lambda a, b: pl.BlockSpec((a, b), lambda i: (0, 0))
    return pl.pallas_call(
        _layer_a_body,
        grid=(GRID,),
        in_specs=[blk(H), blk(H), blk(H), blk(H),
                  full(D, D), full(1, D),
                  full(D, D), full(1, D), full(1, 1)],
        out_specs=[blk(D), full(2, D)],
        out_shape=[jax.ShapeDtypeStruct((N, D), _f32),
                   jax.ShapeDtypeStruct((2, D), _f32)],
    )(h0, h1, a0, a1, w1, b1, w2, b2, eps)


def _layer_b_body(h2_ref, id_ref, st_ref, g_ref, be_ref, o0_ref, o1_ref):
    st = st_ref[...]
    mu = st[0:1, :] * (1.0 / N)
    var = st[1:2, :] * (1.0 / N) - mu * mu
    inv = lax.rsqrt(var + 1e-5) * g_ref[...]
    y = (h2_ref[...] - mu) * inv + be_ref[...] + id_ref[...]
    y = _gelu(y)
    o0_ref[...] = y[:, 0:H]
    o1_ref[...] = y[:, H:D]


def _layer_b(h2, ident, st, gamma, beta):
    blk = lambda w: pl.BlockSpec((R, w), lambda i: (i, 0))
    full = lambda a, b: pl.BlockSpec((a, b), lambda i: (0, 0))
    return pl.pallas_call(
        _layer_b_body,
        grid=(GRID,),
        in_specs=[blk(D), blk(D), full(2, D), full(1, D), full(1, D)],
        out_specs=[blk(H), blk(H)],
        out_shape=[jax.ShapeDtypeStruct((N, H), _f32),
                   jax.ShapeDtypeStruct((N, H), _f32)],
    )(h2, ident, st, gamma, beta)


def _head_body(h0_ref, h1_ref, b_ref, wf1_ref, bf1_ref, wf2_ref, bf2_ref,
               out_ref, sums, cntm):
    i = pl.program_id(0)

    @pl.when(i == 0)
    def _():
        sums[...] = jnp.zeros((G, D), _f32)
        cntm[...] = jnp.zeros((G, H), _f32)

    oh = (b_ref[...] == lax.broadcasted_iota(jnp.int32, (R, G), 1)).astype(_f32)
    dn = (((0,), (0,)), ((), ()))
    sums[:, 0:H] += lax.dot_general(oh, h0_ref[...], dn,
                                    preferred_element_type=_f32)
    sums[:, H:D] += lax.dot_general(oh, h1_ref[...], dn,
                                    preferred_element_type=_f32)
    cntm[...] += lax.dot_general(oh, jnp.ones((R, H), _f32), dn,
                                 preferred_element_type=_f32)

    @pl.when(i == GRID - 1)
    def _():
        cnt = jnp.maximum(cntm[...], 1.0)
        p0 = sums[:, 0:H] / cnt
        p1 = sums[:, H:D] / cnt
        z = (jnp.dot(p0, wf1_ref[0:H, :], preferred_element_type=_f32)
             + jnp.dot(p1, wf1_ref[H:D, :], preferred_element_type=_f32)
             + bf1_ref[...])
        z = _gelu(z)
        out_ref[...] = (jnp.dot(z, wf2_ref[...], preferred_element_type=_f32)
                        + bf2_ref[...])


def _head(h0, h1, batch2, wf1, bf1, wf2, bf2):
    blk = lambda w: pl.BlockSpec((R, w), lambda i: (i, 0))
    full = lambda a, b: pl.BlockSpec((a, b), lambda i: (0, 0))
    return pl.pallas_call(
        _head_body,
        grid=(GRID,),
        in_specs=[blk(H), blk(H), blk(1),
                  full(D, D), full(1, D), full(D, 10), full(1, 10)],
        out_specs=pl.BlockSpec((G, 10), lambda i: (0, 0)),
        out_shape=jax.ShapeDtypeStruct((G, 10), _f32),
        scratch_shapes=[pltpu.VMEM((G, D), _f32), pltpu.VMEM((G, H), _f32)],
    )(h0, h1, batch2, wf1, bf1, wf2, bf2)


def kernel(x, edge_index, batch, params):
    srcp = edge_index[0]
    dstp = edge_index[1]
    batch2 = batch.reshape(N, 1)

    h0 = x[:, 0:H]
    h1 = x[:, H:D]
    for l in range(4):
        g = params[f"gin{l}"]
        bn = params[f"bn{l}"]
        rs = params[f"res{l}"]
        a0, a1 = _get_sc_agg()(h0, h1, srcp, dstp)
        ident = _ident(h0, h1, rs["W"], rs["b"].reshape(1, D))
        h2, st = _layer_a(
            h0, h1, a0, a1,
            g["lin1"]["W"], g["lin1"]["b"].reshape(1, D),
            g["lin2"]["W"], g["lin2"]["b"].reshape(1, D),
            g["eps"].reshape(1, 1))
        h0, h1 = _layer_b(h2, ident, st, bn["gamma"].reshape(1, D),
                          bn["beta"].reshape(1, D))
    return _head(h0, h1, batch2, params["fc1"]["W"],
                 params["fc1"]["b"].reshape(1, D), params["fc2"]["W"],
                 params["fc2"]["b"].reshape(1, 10))


# speedup vs baseline: 1.3467x; 1.1381x over previous
import functools

import jax
import jax.numpy as jnp
from jax import lax
from jax.experimental import pallas as pl
from jax.experimental.pallas import tpu as pltpu
from jax.experimental.pallas import tpu_sc as plsc

N = 10000
D = 256
H = 128
E = 160000
CH = 128
NCHUNK = E // CH
BASE_CHUNKS = NCHUNK // 16
MAX_CHUNKS = BASE_CHUNKS + 1
ACC_ROWS = 10112
ZROWS_PER_TILE = ACC_ROWS // 16
OUT_ROWS_PER_TILE = 632
OUT_TAIL = N - 15 * OUT_ROWS_PER_TILE
G = 64
R = 1000
GRID = N // R

_f32 = jnp.float32


def _sc_agg_body(h0, h1, srcp, dstp, out0, out1,
                 sidx0, didx0, sidx1, didx1, sidx2, didx2, sidx3, didx3,
                 rows0, rows1, zbuf, acc,
                 semi0, semi1, semi2, semi3, semg0, semg1, sems0, sems1):
    c = lax.axis_index("c")
    s = lax.axis_index("s")

    climit = BASE_CHUNKS + jnp.where(s < 2, 1, 0)
    ebase = (BASE_CHUNKS * s + jnp.minimum(s, 2)) * CH

    ibufs = ((sidx0, didx0, semi0), (sidx1, didx1, semi1),
             (sidx2, didx2, semi2), (sidx3, didx3, semi3))
    rbufs = ((rows0, semg0, sems0), (rows1, semg1, sems1))

    def _idx_start(i, p):
        sb, db, semi = ibufs[p]
        off = ebase + i * CH
        pltpu.async_copy(srcp.at[pl.ds(off, CH)], sb, semi)
        pltpu.async_copy(dstp.at[pl.ds(off, CH)], db, semi)

    def _idx_wait(i, p):
        sb, db, semi = ibufs[p]
        off = ebase + i * CH
        pltpu.make_async_copy(srcp.at[pl.ds(off, CH)], sb, semi).wait()
        pltpu.make_async_copy(dstp.at[pl.ds(off, CH)], db, semi).wait()

    def _run(table):
        for p in range(4):
            _idx_start(p, p)

        @pl.loop(0, 64)
        def _zr(i):
            @pl.loop(0, 8)
            def _zc(j):
                zbuf[i, pl.ds(j * 16, 16)] = jnp.zeros((16,), _f32)

        _idx_wait(0, 0)
        pltpu.async_copy(table.at[sidx0], rows0, semg0)
        _idx_wait(1, 1)
        pltpu.async_copy(table.at[sidx1], rows1, semg1)

        zbase = s * ZROWS_PER_TILE
        for k in range(9):
            pltpu.sync_copy(zbuf.at[pl.ds(0, 64)],
                            acc.at[pl.ds(zbase + 64 * k, 64)])
        pltpu.sync_copy(zbuf.at[pl.ds(0, ZROWS_PER_TILE - 576)],
                        acc.at[pl.ds(zbase + 576, ZROWS_PER_TILE - 576)])
        plsc.subcore_barrier()

        @pl.loop(0, MAX_CHUNKS // 4 + 1)
        def _quad(g):
            for p in range(4):
                b = p % 2
                rb, semg, _ = rbufs[b]
                nrb, nsemg, _ = rbufs[1 - b]
                i = 4 * g + p
                sb, db, _ = ibufs[p]

                @pl.when(jnp.logical_and(i + 1 < climit, i > 0))
                def _():
                    nsb, _, _ = ibufs[(p + 1) % 4]
                    _idx_wait(i + 1, (p + 1) % 4)
                    pltpu.async_copy(table.at[nsb], nrb, nsemg)

                @pl.when(i < climit)
                def _():
                    pltpu.make_async_copy(table.at[sb], rb, semg).wait()
                    pltpu.sync_copy(rb, acc.at[db], add=True)

                @pl.when(i + 4 < climit)
                def _():
                    _idx_start(i + 4, p)

    @pl.when(c == 0)
    def _():
        _run(h0)

    @pl.when(c == 1)
    def _():
        _run(h1)

    plsc.subcore_barrier()
    rbase = s * OUT_ROWS_PER_TILE

    def _copy_out(out):
        @pl.when(s < 15)
        def _():
            pltpu.sync_copy(acc.at[pl.ds(rbase, OUT_ROWS_PER_TILE)],
                            out.at[pl.ds(rbase, OUT_ROWS_PER_TILE)])

        @pl.when(s == 15)
        def _():
            pltpu.sync_copy(acc.at[pl.ds(15 * OUT_ROWS_PER_TILE, OUT_TAIL)],
                            out.at[pl.ds(15 * OUT_ROWS_PER_TILE, OUT_TAIL)])

    @pl.when(c == 0)
    def _():
        _copy_out(out0)

    @pl.when(c == 1)
    def _():
        _copy_out(out1)


@functools.lru_cache(maxsize=None)
def _get_sc_agg():
    mesh = plsc.VectorSubcoreMesh(
        core_axis_name="c", subcore_axis_name="s",
        num_cores=2, num_subcores=16)
    return pl.kernel(
        _sc_agg_body,
        out_type=[jax.ShapeDtypeStruct((N, H), _f32),
                  jax.ShapeDtypeStruct((N, H), _f32)],
        mesh=mesh,
        scratch_types=(
            [pltpu.VMEM((CH,), jnp.int32)] * 8
            + [pltpu.VMEM((CH, H), _f32)] * 2
            + [pltpu.VMEM((64, H), _f32)]
            + [pltpu.VMEM_SHARED((ACC_ROWS, H), _f32)]
            + [pltpu.SemaphoreType.DMA] * 8
        ),
    )


def _gelu(y):
    return 0.5 * y * (1.0 + lax.erf(y * 0.7071067811865476))


def _ident_body(h0_ref, h1_ref, wr_ref, br_ref, id_ref):
    id_ref[...] = (jnp.dot(h0_ref[...], wr_ref[0:H, :],
                           preferred_element_type=_f32)
                   + jnp.dot(h1_ref[...], wr_ref[H:D, :],
                             preferred_element_type=_f32)
                   + br_ref[...]).astype(jnp.bfloat16)


def _ident(h0, h1, wr, br):
    blk = lambda w: pl.BlockSpec((R, w), lambda i: (i, 0))
    full = lambda a, b: pl.BlockSpec((a, b), lambda i: (0, 0))
    return pl.pallas_call(
        _ident_body,
        grid=(GRID,),
        in_specs=[blk(H), blk(H), full(D, D), full(1, D)],
        out_specs=blk(D),
        out_shape=jax.ShapeDtypeStruct((N, D), jnp.bfloat16),
    )(h0, h1, wr, br)


def _layer_a_body(h0_ref, h1_ref, a0_ref, a1_ref, w1_ref,
                  b1_ref, w2_ref, b2_ref, eps_ref, h2_ref, st_ref):
    i = pl.program_id(0)
    e1 = 1.0 + eps_ref[0, 0]
    u0 = e1 * h0_ref[...] + a0_ref[...]
    u1 = e1 * h1_ref[...] + a1_ref[...]
    t = (jnp.dot(u0, w1_ref[0:H, :], preferred_element_type=_f32)
         + jnp.dot(u1, w1_ref[H:D, :], preferred_element_type=_f32)
         + b1_ref[...])
    t = jnp.maximum(t, 0.0)
    h2 = jnp.dot(t, w2_ref[...], preferred_element_type=_f32) + b2_ref[...]
    h2_ref[...] = h2

    @pl.when(i == 0)
    def _():
        st_ref[...] = jnp.zeros((2, D), _f32)

    st_ref[0:1, :] += jnp.sum(h2, axis=0, keepdims=True)
    st_ref[1:2, :] += jnp.sum(h2 * h2, axis=0, keepdims=True)


def _layer_a(h0, h1, a0, a1, w1, b1, w2, b2, eps):
    blk = lambda w: pl.BlockSpec((R, w), lambda i: (i, 0))
    full = lambda a, b: pl.BlockSpec((a, b), lambda i: (0, 0))
    return pl.pallas_call(
        _layer_a_body,
        grid=(GRID,),
        in_specs=[blk(H), blk(H), blk(H), blk(H),
                  full(D, D), full(1, D),
                  full(D, D), full(1, D), full(1, 1)],
        out_specs=[blk(D), full(2, D)],
        out_shape=[jax.ShapeDtypeStruct((N, D), _f32),
                   jax.ShapeDtypeStruct((2, D), _f32)],
    )(h0, h1, a0, a1, w1, b1, w2, b2, eps)


def _layer_b_body(h2_ref, id_ref, st_ref, g_ref, be_ref, o0_ref, o1_ref):
    st = st_ref[...]
    mu = st[0:1, :] * (1.0 / N)
    var = st[1:2, :] * (1.0 / N) - mu * mu
    inv = lax.rsqrt(var + 1e-5) * g_ref[...]
    y = ((h2_ref[...] - mu) * inv + be_ref[...]
         + id_ref[...].astype(_f32))
    y = _gelu(y)
    o0_ref[...] = y[:, 0:H]
    o1_ref[...] = y[:, H:D]


def _layer_b(h2, ident, st, gamma, beta):
    blk = lambda w: pl.BlockSpec((R, w), lambda i: (i, 0))
    full = lambda a, b: pl.BlockSpec((a, b), lambda i: (0, 0))
    return pl.pallas_call(
        _layer_b_body,
        grid=(GRID,),
        in_specs=[blk(D), blk(D), full(2, D), full(1, D), full(1, D)],
        out_specs=[blk(H), blk(H)],
        out_shape=[jax.ShapeDtypeStruct((N, H), _f32),
                   jax.ShapeDtypeStruct((N, H), _f32)],
    )(h2, ident, st, gamma, beta)


def _head_body(h0_ref, h1_ref, b_ref, wf1_ref, bf1_ref, wf2_ref, bf2_ref,
               out_ref, sums, cntm):
    i = pl.program_id(0)

    @pl.when(i == 0)
    def _():
        sums[...] = jnp.zeros((G, D), _f32)
        cntm[...] = jnp.zeros((G, H), _f32)

    oh = (b_ref[...] == lax.broadcasted_iota(jnp.int32, (R, G), 1)).astype(_f32)
    dn = (((0,), (0,)), ((), ()))
    sums[:, 0:H] += lax.dot_general(oh, h0_ref[...], dn,
                                    preferred_element_type=_f32)
    sums[:, H:D] += lax.dot_general(oh, h1_ref[...], dn,
                                    preferred_element_type=_f32)
    cntm[...] += lax.dot_general(oh, jnp.ones((R, H), _f32), dn,
                                 preferred_element_type=_f32)

    @pl.when(i == GRID - 1)
    def _():
        cnt = jnp.maximum(cntm[...], 1.0)
        p0 = sums[:, 0:H] / cnt
        p1 = sums[:, H:D] / cnt
        z = (jnp.dot(p0, wf1_ref[0:H, :], preferred_element_type=_f32)
             + jnp.dot(p1, wf1_ref[H:D, :], preferred_element_type=_f32)
             + bf1_ref[...])
        z = _gelu(z)
        out_ref[...] = (jnp.dot(z, wf2_ref[...], preferred_element_type=_f32)
                        + bf2_ref[...])


def _head(h0, h1, batch2, wf1, bf1, wf2, bf2):
    blk = lambda w: pl.BlockSpec((R, w), lambda i: (i, 0))
    full = lambda a, b: pl.BlockSpec((a, b), lambda i: (0, 0))
    return pl.pallas_call(
        _head_body,
        grid=(GRID,),
        in_specs=[blk(H), blk(H), blk(1),
                  full(D, D), full(1, D), full(D, 10), full(1, 10)],
        out_specs=pl.BlockSpec((G, 10), lambda i: (0, 0)),
        out_shape=jax.ShapeDtypeStruct((G, 10), _f32),
        scratch_shapes=[pltpu.VMEM((G, D), _f32), pltpu.VMEM((G, H), _f32)],
    )(h0, h1, batch2, wf1, bf1, wf2, bf2)


def kernel(x, edge_index, batch, params):
    srcp = edge_index[0]
    dstp = edge_index[1]
    batch2 = batch.reshape(N, 1)

    h0 = x[:, 0:H]
    h1 = x[:, H:D]
    for l in range(4):
        g = params[f"gin{l}"]
        bn = params[f"bn{l}"]
        rs = params[f"res{l}"]
        a0, a1 = _get_sc_agg()(h0, h1, srcp, dstp)
        ident = _ident(h0, h1, rs["W"], rs["b"].reshape(1, D))
        h2, st = _layer_a(
            h0, h1, a0, a1,
            g["lin1"]["W"], g["lin1"]["b"].reshape(1, D),
            g["lin2"]["W"], g["lin2"]["b"].reshape(1, D),
            g["eps"].reshape(1, 1))
        h0, h1 = _layer_b(h2, ident, st, bn["gamma"].reshape(1, D),
                          bn["beta"].reshape(1, D))
    return _head(h0, h1, batch2, params["fc1"]["W"],
                 params["fc1"]["b"].reshape(1, D), params["fc2"]["W"],
                 params["fc2"]["b"].reshape(1, 10))


# speedup vs baseline: 1.4651x; 1.0879x over previous
import functools

import jax
import jax.numpy as jnp
from jax import lax
from jax.experimental import pallas as pl
from jax.experimental.pallas import tpu as pltpu
from jax.experimental.pallas import tpu_sc as plsc

N = 10000
D = 256
H = 128
E = 160000
CH = 128
NCHUNK = E // CH
BASE_CHUNKS = NCHUNK // 16
MAX_CHUNKS = BASE_CHUNKS + 1
ACC_ROWS = 10112
ZROWS_PER_TILE = ACC_ROWS // 16
OUT_ROWS_PER_TILE = 632
OUT_TAIL = N - 15 * OUT_ROWS_PER_TILE
G = 64
R = 1000
GRID = N // R

_f32 = jnp.float32


def _sc_agg_body(h0, h1, srcp, dstp, out0, out1,
                 sidx0, didx0, sidx1, didx1, sidx2, didx2, sidx3, didx3,
                 rows0, rows1, rows2, acc,
                 semi0, semi1, semi2, semi3, semg0, semg1, semg2):
    c = lax.axis_index("c")
    s = lax.axis_index("s")

    climit = BASE_CHUNKS + jnp.where(s < 2, 1, 0)
    ebase = (BASE_CHUNKS * s + jnp.minimum(s, 2)) * CH

    ibufs = ((sidx0, didx0, semi0), (sidx1, didx1, semi1),
             (sidx2, didx2, semi2), (sidx3, didx3, semi3))
    rbufs = ((rows0, semg0), (rows1, semg1), (rows2, semg2))

    def _idx_start(i, p):
        sb, db, semi = ibufs[p]
        off = ebase + i * CH
        pltpu.async_copy(srcp.at[pl.ds(off, CH)], sb, semi)
        pltpu.async_copy(dstp.at[pl.ds(off, CH)], db, semi)

    def _idx_wait(i, p):
        sb, db, semi = ibufs[p]
        off = ebase + i * CH
        pltpu.make_async_copy(srcp.at[pl.ds(off, CH)], sb, semi).wait()
        pltpu.make_async_copy(dstp.at[pl.ds(off, CH)], db, semi).wait()

    def _run(table):
        for p in range(4):
            _idx_start(p, p)
        _idx_wait(0, 0)
        pltpu.async_copy(table.at[sidx0], rows0, semg0)
        _idx_wait(1, 1)
        pltpu.async_copy(table.at[sidx1], rows1, semg1)

        @pl.loop(0, 64)
        def _zr(i):
            @pl.loop(0, 8)
            def _zc(j):
                rows2[i, pl.ds(j * 16, 16)] = jnp.zeros((16,), _f32)

        zbase = s * ZROWS_PER_TILE
        for k in range(9):
            pltpu.sync_copy(rows2.at[pl.ds(0, 64)],
                            acc.at[pl.ds(zbase + 64 * k, 64)])
        pltpu.sync_copy(rows2.at[pl.ds(0, ZROWS_PER_TILE - 576)],
                        acc.at[pl.ds(zbase + 576, ZROWS_PER_TILE - 576)])
        plsc.subcore_barrier()

        @pl.loop(0, MAX_CHUNKS // 12 + 1)
        def _twelve(g):
            for k in range(12):
                i = 12 * g + k
                p4 = k % 4
                p3 = k % 3
                sb, db, _ = ibufs[p4]
                rb, semg = rbufs[p3]

                @pl.when(i + 2 < climit)
                def _():
                    nsb, _, _ = ibufs[(p4 + 2) % 4]
                    nrb, nsemg = rbufs[(p3 + 2) % 3]
                    _idx_wait(i + 2, (p4 + 2) % 4)
                    pltpu.async_copy(table.at[nsb], nrb, nsemg)

                @pl.when(i < climit)
                def _():
                    pltpu.make_async_copy(table.at[sb], rb, semg).wait()
                    pltpu.sync_copy(rb, acc.at[db], add=True)

                @pl.when(i + 4 < climit)
                def _():
                    _idx_start(i + 4, p4)

    @pl.when(c == 0)
    def _():
        _run(h0)

    @pl.when(c == 1)
    def _():
        _run(h1)

    plsc.subcore_barrier()
    rbase = s * OUT_ROWS_PER_TILE

    def _copy_out(out):
        @pl.when(s < 15)
        def _():
            pltpu.sync_copy(acc.at[pl.ds(rbase, OUT_ROWS_PER_TILE)],
                            out.at[pl.ds(rbase, OUT_ROWS_PER_TILE)])

        @pl.when(s == 15)
        def _():
            pltpu.sync_copy(acc.at[pl.ds(15 * OUT_ROWS_PER_TILE, OUT_TAIL)],
                            out.at[pl.ds(15 * OUT_ROWS_PER_TILE, OUT_TAIL)])

    @pl.when(c == 0)
    def _():
        _copy_out(out0)

    @pl.when(c == 1)
    def _():
        _copy_out(out1)


@functools.lru_cache(maxsize=None)
def _get_sc_agg():
    mesh = plsc.VectorSubcoreMesh(
        core_axis_name="c", subcore_axis_name="s",
        num_cores=2, num_subcores=16)
    return pl.kernel(
        _sc_agg_body,
        out_type=[jax.ShapeDtypeStruct((N, H), _f32),
                  jax.ShapeDtypeStruct((N, H), _f32)],
        mesh=mesh,
        scratch_types=(
            [pltpu.VMEM((CH,), jnp.int32)] * 8
            + [pltpu.VMEM((CH, H), _f32)] * 3
            + [pltpu.VMEM_SHARED((ACC_ROWS, H), _f32)]
            + [pltpu.SemaphoreType.DMA] * 7
        ),
    )


def _gelu(y):
    return 0.5 * y * (1.0 + lax.erf(y * 0.7071067811865476))


def _ident_body(h0_ref, h1_ref, wr_ref, br_ref, id_ref):
    id_ref[...] = (jnp.dot(h0_ref[...], wr_ref[0:H, :],
                           preferred_element_type=_f32)
                   + jnp.dot(h1_ref[...], wr_ref[H:D, :],
                             preferred_element_type=_f32)
                   + br_ref[...]).astype(jnp.bfloat16)


def _ident(h0, h1, wr, br):
    blk = lambda w: pl.BlockSpec((R, w), lambda i: (i, 0))
    full = lambda a, b: pl.BlockSpec((a, b), lambda i: (0, 0))
    return pl.pallas_call(
        _ident_body,
        grid=(GRID,),
        in_specs=[blk(H), blk(H), full(D, D), full(1, D)],
        out_specs=blk(D),
        out_shape=jax.ShapeDtypeStruct((N, D), jnp.bfloat16),
    )(h0, h1, wr, br)


def _layer_a_body(h0_ref, h1_ref, a0_ref, a1_ref, w1_ref,
                  b1_ref, w2_ref, b2_ref, eps_ref, h2_ref, st_ref):
    i = pl.program_id(0)
    e1 = 1.0 + eps_ref[0, 0]
    u0 = e1 * h0_ref[...] + a0_ref[...]
    u1 = e1 * h1_ref[...] + a1_ref[...]
    t = (jnp.dot(u0, w1_ref[0:H, :], preferred_element_type=_f32)
         + jnp.dot(u1, w1_ref[H:D, :], preferred_element_type=_f32)
         + b1_ref[...])
    t = jnp.maximum(t, 0.0)
    h2 = jnp.dot(t, w2_ref[...], preferred_element_type=_f32) + b2_ref[...]
    h2_ref[...] = h2

    @pl.when(i == 0)
    def _():
        st_ref[...] = jnp.zeros((2, D), _f32)

    st_ref[0:1, :] += jnp.sum(h2, axis=0, keepdims=True)
    st_ref[1:2, :] += jnp.sum(h2 * h2, axis=0, keepdims=True)


def _layer_a(h0, h1, a0, a1, w1, b1, w2, b2, eps):
    blk = lambda w: pl.BlockSpec((R, w), lambda i: (i, 0))
    full = lambda a, b: pl.BlockSpec((a, b), lambda i: (0, 0))
    return pl.pallas_call(
        _layer_a_body,
        grid=(GRID,),
        in_specs=[blk(H), blk(H), blk(H), blk(H),
                  full(D, D), full(1, D),
                  full(D, D), full(1, D), full(1, 1)],
        out_specs=[blk(D), full(2, D)],
        out_shape=[jax.ShapeDtypeStruct((N, D), _f32),
                   jax.ShapeDtypeStruct((2, D), _f32)],
    )(h0, h1, a0, a1, w1, b1, w2, b2, eps)


def _layer_b_body(h2_ref, id_ref, st_ref, g_ref, be_ref, o0_ref, o1_ref):
    st = st_ref[...]
    mu = st[0:1, :] * (1.0 / N)
    var = st[1:2, :] * (1.0 / N) - mu * mu
    inv = lax.rsqrt(var + 1e-5) * g_ref[...]
    y = ((h2_ref[...] - mu) * inv + be_ref[...]
         + id_ref[...].astype(_f32))
    y = _gelu(y)
    o0_ref[...] = y[:, 0:H]
    o1_ref[...] = y[:, H:D]


def _layer_b(h2, ident, st, gamma, beta):
    blk = lambda w: pl.BlockSpec((R, w), lambda i: (i, 0))
    full = lambda a, b: pl.BlockSpec((a, b), lambda i: (0, 0))
    return pl.pallas_call(
        _layer_b_body,
        grid=(GRID,),
        in_specs=[blk(D), blk(D), full(2, D), full(1, D), full(1, D)],
        out_specs=[blk(H), blk(H)],
        out_shape=[jax.ShapeDtypeStruct((N, H), _f32),
                   jax.ShapeDtypeStruct((N, H), _f32)],
    )(h2, ident, st, gamma, beta)


def _head_body(h0_ref, h1_ref, b_ref, wf1_ref, bf1_ref, wf2_ref, bf2_ref,
               out_ref, sums, cntm):
    i = pl.program_id(0)

    @pl.when(i == 0)
    def _():
        sums[...] = jnp.zeros((G, D), _f32)
        cntm[...] = jnp.zeros((G, H), _f32)

    oh = (b_ref[...] == lax.broadcasted_iota(jnp.int32, (R, G), 1)).astype(_f32)
    dn = (((0,), (0,)), ((), ()))
    sums[:, 0:H] += lax.dot_general(oh, h0_ref[...], dn,
                                    preferred_element_type=_f32)
    sums[:, H:D] += lax.dot_general(oh, h1_ref[...], dn,
                                    preferred_element_type=_f32)
    cntm[...] += lax.dot_general(oh, jnp.ones((R, H), _f32), dn,
                                 preferred_element_type=_f32)

    @pl.when(i == GRID - 1)
    def _():
        cnt = jnp.maximum(cntm[...], 1.0)
        p0 = sums[:, 0:H] / cnt
        p1 = sums[:, H:D] / cnt
        z = (jnp.dot(p0, wf1_ref[0:H, :], preferred_element_type=_f32)
             + jnp.dot(p1, wf1_ref[H:D, :], preferred_element_type=_f32)
             + bf1_ref[...])
        z = _gelu(z)
        out_ref[...] = (jnp.dot(z, wf2_ref[...], preferred_element_type=_f32)
                        + bf2_ref[...])


def _head(h0, h1, batch2, wf1, bf1, wf2, bf2):
    blk = lambda w: pl.BlockSpec((R, w), lambda i: (i, 0))
    full = lambda a, b: pl.BlockSpec((a, b), lambda i: (0, 0))
    return pl.pallas_call(
        _head_body,
        grid=(GRID,),
        in_specs=[blk(H), blk(H), blk(1),
                  full(D, D), full(1, D), full(D, 10), full(1, 10)],
        out_specs=pl.BlockSpec((G, 10), lambda i: (0, 0)),
        out_shape=jax.ShapeDtypeStruct((G, 10), _f32),
        scratch_shapes=[pltpu.VMEM((G, D), _f32), pltpu.VMEM((G, H), _f32)],
    )(h0, h1, batch2, wf1, bf1, wf2, bf2)


def kernel(x, edge_index, batch, params):
    srcp = edge_index[0]
    dstp = edge_index[1]
    batch2 = batch.reshape(N, 1)

    h0 = x[:, 0:H]
    h1 = x[:, H:D]
    for l in range(4):
        g = params[f"gin{l}"]
        bn = params[f"bn{l}"]
        rs = params[f"res{l}"]
        a0, a1 = _get_sc_agg()(h0, h1, srcp, dstp)
        ident = _ident(h0, h1, rs["W"], rs["b"].reshape(1, D))
        h2, st = _layer_a(
            h0, h1, a0, a1,
            g["lin1"]["W"], g["lin1"]["b"].reshape(1, D),
            g["lin2"]["W"], g["lin2"]["b"].reshape(1, D),
            g["eps"].reshape(1, 1))
        h0, h1 = _layer_b(h2, ident, st, bn["gamma"].reshape(1, D),
                          bn["beta"].reshape(1, D))
    return _head(h0, h1, batch2, params["fc1"]["W"],
                 params["fc1"]["b"].reshape(1, D), params["fc2"]["W"],
                 params["fc2"]["b"].reshape(1, 10))
